# baseline (speedup 1.0000x reference)
import functools

import jax
import jax.numpy as jnp
from jax import lax
from jax.experimental import pallas as pl
from jax.experimental.pallas import tpu as pltpu
from jax.experimental.pallas import tpu_sc as plsc


H, W = 384, 384
L = 16
NC, NS = 2, 16
NW = NC * NS
ROWS_PER_CHUNK = 48
BUF = ROWS_PER_CHUNK * W
NBUF = 3


def _sc_body(x_hbm, o_hbm, *refs):
    bufs_in = refs[0:NBUF]
    bufs_out = refs[NBUF:2 * NBUF]
    sems_in = refs[2 * NBUF:3 * NBUF]
    sems_out = refs[3 * NBUF:4 * NBUF]

    n = x_hbm.shape[0]
    per_w = n // NW
    chunks = per_w // BUF
    wid = lax.axis_index("s") * NC + lax.axis_index("c")
    base = wid * per_w

    pat_even = (lax.iota(jnp.int32, L) % 2).astype(jnp.float32)
    pat_odd = 1.0 - pat_even

    def in_dma(i, b):
        pltpu.async_copy(
            x_hbm.at[pl.ds(base + i * BUF, BUF)], bufs_in[b], sems_in[b])

    def out_dma(i, b):
        pltpu.async_copy(
            bufs_out[b], o_hbm.at[pl.ds(base + i * BUF, BUF)], sems_out[b])

    def wait_in(b):
        pltpu.make_async_copy(
            x_hbm.at[pl.ds(base, BUF)], bufs_in[b], sems_in[b]).wait()

    def wait_out(b):
        pltpu.make_async_copy(
            bufs_out[b], o_hbm.at[pl.ds(base, BUF)], sems_out[b]).wait()

    for b in range(NBUF):
        in_dma(b, b)

    def compute(b):
        src, dst = bufs_in[b], bufs_out[b]

        def rowpair(j, _):
            off = j * (2 * W)
            for t in range(W // L):
                s = off + t * L
                dst[pl.ds(s, L)] = src[pl.ds(s, L)] * pat_even
            for t in range(W // L):
                s = off + W + t * L
                dst[pl.ds(s, L)] = src[pl.ds(s, L)] * pat_odd
            return 0

        lax.fori_loop(0, ROWS_PER_CHUNK // 2, rowpair, 0)

    def step(g, _):
        for b in range(NBUF):
            i = g + b

            @pl.when(i >= NBUF)
            def _():
                wait_out(b)
            wait_in(b)
            compute(b)
            out_dma(i, b)

            @pl.when(i + NBUF < chunks)
            def _():
                in_dma(i + NBUF, b)
        return 0

    lax.fori_loop(0, chunks // NBUF, lambda g, c: step(g * NBUF, c), 0)

    for b in range(NBUF):
        wait_out(b)


def _sc_mul(xf):
    n = xf.shape[0]
    mesh = plsc.VectorSubcoreMesh(core_axis_name="c", subcore_axis_name="s")
    scratch = (
        [pltpu.VMEM((BUF,), jnp.float32) for _ in range(2 * NBUF)]
        + [pltpu.SemaphoreType.DMA for _ in range(2 * NBUF)]
    )
    f = functools.partial(
        pl.kernel,
        mesh=mesh,
        out_type=jax.ShapeDtypeStruct((n,), jnp.float32),
        scratch_types=scratch,
    )(_sc_body)
    return f(xf)


def kernel(x, mask):
    B, C = x.shape[0], x.shape[1]
    out = _sc_mul(x.reshape(-1))
    return out.reshape(B, C, H, W)


# speedup vs baseline: 1.0791x; 1.0791x over previous
import functools

import jax
import jax.numpy as jnp
from jax import lax
from jax.experimental import pallas as pl
from jax.experimental.pallas import tpu as pltpu
from jax.experimental.pallas import tpu_sc as plsc


H, W = 384, 384
PLANE = H * W
N_PLANES = 768
TC_PLANES = 576
SC_PLANES = N_PLANES - TC_PLANES

L = 16
NC, NS = 2, 16
NW = NC * NS
ROWS_PER_CHUNK = 48
BUF = ROWS_PER_CHUNK * W
NBUF = 3

PLANES_PER_BLOCK = 8


def _tc_body(x_ref, m_ref, o_ref):
    o_ref[...] = x_ref[...] * m_ref[None]


def _tc_mul(xf, mask):
    return pl.pallas_call(
        _tc_body,
        grid=(TC_PLANES // PLANES_PER_BLOCK,),
        in_specs=[
            pl.BlockSpec((PLANES_PER_BLOCK, H, W), lambda i: (i, 0, 0)),
            pl.BlockSpec((H, W), lambda i: (0, 0)),
        ],
        out_specs=pl.BlockSpec((PLANES_PER_BLOCK, H, W), lambda i: (i, 0, 0)),
        out_shape=jax.ShapeDtypeStruct((TC_PLANES, H, W), xf.dtype),
    )(xf, mask)


def _sc_body(x_hbm, o_hbm, *refs):
    bufs_in = refs[0:NBUF]
    bufs_out = refs[NBUF:2 * NBUF]
    sems_in = refs[2 * NBUF:3 * NBUF]
    sems_out = refs[3 * NBUF:4 * NBUF]

    n = o_hbm.shape[0]
    per_w = n // NW
    chunks = per_w // BUF
    wid = lax.axis_index("s") * NC + lax.axis_index("c")
    base_in = (x_hbm.shape[0] - n) + wid * per_w
    base_out = wid * per_w

    pat_even = (lax.iota(jnp.int32, L) % 2).astype(jnp.float32)
    pat_odd = 1.0 - pat_even

    def in_dma(i, b):
        pltpu.async_copy(
            x_hbm.at[pl.ds(base_in + i * BUF, BUF)], bufs_in[b], sems_in[b])

    def out_dma(i, b):
        pltpu.async_copy(
            bufs_out[b], o_hbm.at[pl.ds(base_out + i * BUF, BUF)], sems_out[b])

    def wait_in(b):
        pltpu.make_async_copy(
            x_hbm.at[pl.ds(base_in, BUF)], bufs_in[b], sems_in[b]).wait()

    def wait_out(b):
        pltpu.make_async_copy(
            bufs_out[b], o_hbm.at[pl.ds(base_out, BUF)], sems_out[b]).wait()

    for b in range(NBUF):
        in_dma(b, b)

    def compute(b):
        src, dst = bufs_in[b], bufs_out[b]

        def rowpair(j, _):
            off = j * (2 * W)
            for t in range(W // L):
                s = off + t * L
                dst[pl.ds(s, L)] = src[pl.ds(s, L)] * pat_even
            for t in range(W // L):
                s = off + W + t * L
                dst[pl.ds(s, L)] = src[pl.ds(s, L)] * pat_odd
            return 0

        lax.fori_loop(0, ROWS_PER_CHUNK // 2, rowpair, 0)

    def step(g, _):
        for b in range(NBUF):
            i = g + b

            @pl.when(i >= NBUF)
            def _():
                wait_out(b)
            wait_in(b)
            compute(b)
            out_dma(i, b)

            @pl.when(i + NBUF < chunks)
            def _():
                in_dma(i + NBUF, b)
        return 0

    lax.fori_loop(0, chunks // NBUF, lambda g, c: step(g * NBUF, c), 0)

    for b in range(NBUF):
        wait_out(b)


def _sc_mul(x_flat, n_out):
    mesh = plsc.VectorSubcoreMesh(core_axis_name="c", subcore_axis_name="s")
    scratch = (
        [pltpu.VMEM((BUF,), jnp.float32) for _ in range(2 * NBUF)]
        + [pltpu.SemaphoreType.DMA for _ in range(2 * NBUF)]
    )
    f = functools.partial(
        pl.kernel,
        mesh=mesh,
        out_type=jax.ShapeDtypeStruct((n_out,), jnp.float32),
        scratch_types=scratch,
    )(_sc_body)
    return f(x_flat)


def kernel(x, mask):
    B, C = x.shape[0], x.shape[1]
    xf = x.reshape(N_PLANES, H, W)
    tc_out = _tc_mul(xf, mask)
    sc_out = _sc_mul(x.reshape(-1), SC_PLANES * PLANE)
    out = jnp.concatenate([tc_out, sc_out.reshape(SC_PLANES, H, W)], axis=0)
    return out.reshape(B, C, H, W)


# speedup vs baseline: 2.1493x; 1.9917x over previous
import jax
import jax.numpy as jnp
from jax.experimental import pallas as pl


H, W = 384, 384
N_PLANES = 768
SPLIT = 576
PLANES_PER_BLOCK = 8


def _tc_body(x_ref, m_ref, o_ref):
    o_ref[...] = x_ref[...] * m_ref[None]


def _tc_mul(xf, mask, start, count):
    return pl.pallas_call(
        _tc_body,
        grid=(count // PLANES_PER_BLOCK,),
        in_specs=[
            pl.BlockSpec((PLANES_PER_BLOCK, H, W),
                         lambda i: (start // PLANES_PER_BLOCK + i, 0, 0)),
            pl.BlockSpec((H, W), lambda i: (0, 0)),
        ],
        out_specs=pl.BlockSpec((PLANES_PER_BLOCK, H, W), lambda i: (i, 0, 0)),
        out_shape=jax.ShapeDtypeStruct((count, H, W), xf.dtype),
    )(xf, mask)


def kernel(x, mask):
    B, C = x.shape[0], x.shape[1]
    xf = x.reshape(N_PLANES, H, W)
    a = _tc_mul(xf, mask, 0, SPLIT)
    b = _tc_mul(xf, mask, SPLIT, N_PLANES - SPLIT)
    return jnp.concatenate([a, b], axis=0).reshape(B, C, H, W)


# speedup vs baseline: 4.3341x; 2.0165x over previous
import jax
import jax.numpy as jnp
from jax.experimental import pallas as pl


H, W = 384, 384
PLANES_PER_BLOCK = 16


def _body(x_ref, m_ref, o_ref):
    o_ref[...] = x_ref[...] * m_ref[None]


def kernel(x, mask):
    B, C = x.shape[0], x.shape[1]
    n_planes = B * C
    xf = x.reshape(n_planes, H, W)
    out = pl.pallas_call(
        _body,
        grid=(n_planes // PLANES_PER_BLOCK,),
        in_specs=[
            pl.BlockSpec((PLANES_PER_BLOCK, H, W), lambda i: (i, 0, 0)),
            pl.BlockSpec((H, W), lambda i: (0, 0)),
        ],
        out_specs=pl.BlockSpec((PLANES_PER_BLOCK, H, W), lambda i: (i, 0, 0)),
        out_shape=jax.ShapeDtypeStruct((n_planes, H, W), x.dtype),
    )(xf, mask)
    return out.reshape(B, C, H, W)


# speedup vs baseline: 4.3547x; 1.0048x over previous
import jax
import jax.numpy as jnp
from jax.experimental import pallas as pl


H, W = 384, 384
PLANES_PER_BLOCK = 24


def _body(x_ref, m_ref, o_ref):
    o_ref[...] = x_ref[...] * m_ref[None]


def kernel(x, mask):
    B, C = x.shape[0], x.shape[1]
    n_planes = B * C
    xf = x.reshape(n_planes, H, W)
    out = pl.pallas_call(
        _body,
        grid=(n_planes // PLANES_PER_BLOCK,),
        in_specs=[
            pl.BlockSpec((PLANES_PER_BLOCK, H, W), lambda i: (i, 0, 0)),
            pl.BlockSpec((H, W), lambda i: (0, 0)),
        ],
        out_specs=pl.BlockSpec((PLANES_PER_BLOCK, H, W), lambda i: (i, 0, 0)),
        out_shape=jax.ShapeDtypeStruct((n_planes, H, W), x.dtype),
    )(xf, mask)
    return out.reshape(B, C, H, W)
